# Initial kernel scaffold; baseline (speedup 1.0000x reference)
import jax
import jax.numpy as jnp
from jax.experimental import pallas as pl


def kernel(ze, codebook):
    raise NotImplementedError("write your pallas kernel here")


# speedup vs baseline: 1.5773x; 1.5773x over previous
import functools

import jax
import jax.numpy as jnp
from jax.experimental import pallas as pl

NUM_CODES = 1024
DIM = 64
DEAD_THRESHOLD = 2.0
N_TOKENS = 16 * 32 * 32
HW = 32 * 32
BPB = 2
BLK = BPB * HW
NB = N_TOKENS // BLK


def _vq_kernel(ze_ref, cb_ref, zq_ref, idx_ref,
               loss_ref, usage_ref, ppl_ref, dead_ref):
    i = pl.program_id(0)
    nb = pl.num_programs(0)

    zr = ze_ref[...]
    cb = cb_ref[...]

    zcm = jnp.concatenate([zr[b] for b in range(BPB)], axis=1)
    flat = zcm.T

    z2 = jnp.sum(flat * flat, axis=1)
    cb2 = jnp.sum(cb * cb, axis=1)

    zc = jax.lax.dot_general(
        flat, cb, (((1,), (1,)), ((), ())),
        preferred_element_type=jnp.float32,
        precision=jax.lax.Precision.DEFAULT)
    d = (z2[:, None] - 2.0 * zc) + cb2[None, :]

    mind = jnp.min(d, axis=1, keepdims=True)
    cidx = jax.lax.broadcasted_iota(jnp.int32, (BLK, NUM_CODES), 1)
    idx = jnp.min(jnp.where(d == mind, cidx, NUM_CODES), axis=1)
    idx = idx.astype(jnp.int32)

    oh_bool = (jax.lax.broadcasted_iota(jnp.int32, (BLK, NUM_CODES), 1)
               == idx[:, None])
    oh_bf = oh_bool.astype(jnp.bfloat16)
    oh_f = oh_bool.astype(jnp.float32)

    cb_hi = cb.astype(jnp.bfloat16)
    cb_lo = (cb - cb_hi.astype(jnp.float32)).astype(jnp.bfloat16)
    dn = (((0,), (1,)), ((), ()))
    zq_cm = (jax.lax.dot_general(cb_hi, oh_bf, dn,
                                 preferred_element_type=jnp.float32)
             + jax.lax.dot_general(cb_lo, oh_bf, dn,
                                   preferred_element_type=jnp.float32))
    diff = zq_cm - zcm
    for b in range(BPB):
        zq_ref[b] = zq_cm[:, b * HW:(b + 1) * HW]
    idx_ref[0, 0, :] = idx

    bl = jnp.sum(diff * diff).reshape(1, 1)
    uc = jnp.sum(oh_f, axis=0, keepdims=True)

    @pl.when(i == 0)
    def _init():
        loss_ref[...] = bl
        usage_ref[...] = uc

    @pl.when(i != 0)
    def _acc():
        loss_ref[...] += bl
        usage_ref[...] += uc

    @pl.when(i == nb - 1)
    def _finalize():
        counts = usage_ref[...]
        loss_ref[...] = loss_ref[...] * (1.0 / (N_TOKENS * DIM))
        dead_ref[...] = jnp.sum((counts < DEAD_THRESHOLD).astype(jnp.int32),
                                axis=1, keepdims=True)
        usage = counts * (1.0 / N_TOKENS)
        ppl_ref[...] = jnp.exp(-jnp.sum(usage * jnp.log(usage + 1e-10),
                                        axis=1, keepdims=True))
        usage_ref[...] = usage


@functools.partial(jax.jit, static_argnames=("interpret",))
def kernel(ze, codebook, interpret=False):
    b, c, h, w = ze.shape
    ze_r = ze.reshape(b, c, h * w)

    out_shapes = (
        jax.ShapeDtypeStruct((b, c, h * w), jnp.float32),
        jax.ShapeDtypeStruct((NB, 1, BLK), jnp.int32),
        jax.ShapeDtypeStruct((1, 1), jnp.float32),
        jax.ShapeDtypeStruct((1, NUM_CODES), jnp.float32),
        jax.ShapeDtypeStruct((1, 1), jnp.float32),
        jax.ShapeDtypeStruct((1, 1), jnp.int32),
    )
    zq, idx, loss, usage, ppl, dead = pl.pallas_call(
        _vq_kernel,
        grid=(NB,),
        in_specs=[
            pl.BlockSpec((BPB, c, h * w), lambda i: (i, 0, 0)),
            pl.BlockSpec((NUM_CODES, DIM), lambda i: (0, 0)),
        ],
        out_specs=(
            pl.BlockSpec((BPB, c, h * w), lambda i: (i, 0, 0)),
            pl.BlockSpec((1, 1, BLK), lambda i: (i, 0, 0)),
            pl.BlockSpec((1, 1), lambda i: (0, 0)),
            pl.BlockSpec((1, NUM_CODES), lambda i: (0, 0)),
            pl.BlockSpec((1, 1), lambda i: (0, 0)),
            pl.BlockSpec((1, 1), lambda i: (0, 0)),
        ),
        out_shape=out_shapes,
        interpret=interpret,
    )(ze_r, codebook)

    zq_st = zq.reshape(b, c, h, w)
    indices = idx.reshape(b, h, w)
    loss = loss.reshape(())
    ppl = ppl.reshape(())
    dead = dead.reshape(())
    usage = usage.reshape(NUM_CODES)
    return (zq_st, indices, loss, loss, ppl, dead, usage)


# speedup vs baseline: 1.6065x; 1.0185x over previous
import functools

import jax
import jax.numpy as jnp
from jax.experimental import pallas as pl

NUM_CODES = 1024
DIM = 64
DEAD_THRESHOLD = 2.0
N_TOKENS = 16 * 32 * 32
BLK = 2048
NB = N_TOKENS // BLK


def _vq_kernel(flat_ref, cb_ref, zq_ref, idx_ref,
               loss_ref, usage_ref, ppl_ref, dead_ref):
    i = pl.program_id(0)
    nb = pl.num_programs(0)

    flat = flat_ref[...]
    cb = cb_ref[...]
    z2 = jnp.sum(flat * flat, axis=1)
    cb2 = jnp.sum(cb * cb, axis=1)

    zc = jax.lax.dot_general(
        flat, cb, (((1,), (1,)), ((), ())),
        preferred_element_type=jnp.float32,
        precision=jax.lax.Precision.DEFAULT)
    d = (z2[:, None] - 2.0 * zc) + cb2[None, :]

    mind = jnp.min(d, axis=1, keepdims=True)
    cidx = jax.lax.broadcasted_iota(jnp.int32, (BLK, NUM_CODES), 1)
    idx = jnp.min(jnp.where(d == mind, cidx, NUM_CODES), axis=1)
    idx = idx.astype(jnp.int32)

    oh_bf = (cidx == idx[:, None]).astype(jnp.bfloat16)

    cb_hi = cb.astype(jnp.bfloat16)
    cb_lo = (cb - cb_hi.astype(jnp.float32)).astype(jnp.bfloat16)
    dn = (((1,), (0,)), ((), ()))
    zq = (jax.lax.dot_general(oh_bf, cb_hi, dn,
                              preferred_element_type=jnp.float32)
          + jax.lax.dot_general(oh_bf, cb_lo, dn,
                                preferred_element_type=jnp.float32))
    zq_ref[...] = zq
    idx_ref[0, 0, :] = idx

    bl = jnp.sum(mind).reshape(1, 1)
    ones = jnp.ones((8, BLK), dtype=jnp.bfloat16)
    uc8 = jax.lax.dot_general(ones, oh_bf, (((1,), (0,)), ((), ())),
                              preferred_element_type=jnp.float32)
    uc = uc8[0:1, :]

    @pl.when(i == 0)
    def _init():
        loss_ref[...] = bl
        usage_ref[...] = uc

    @pl.when(i != 0)
    def _acc():
        loss_ref[...] += bl
        usage_ref[...] += uc

    @pl.when(i == nb - 1)
    def _finalize():
        counts = usage_ref[...]
        loss_ref[...] = loss_ref[...] * (1.0 / (N_TOKENS * DIM))
        dead_ref[...] = jnp.sum((counts < DEAD_THRESHOLD).astype(jnp.int32),
                                axis=1, keepdims=True)
        usage = counts * (1.0 / N_TOKENS)
        ppl_ref[...] = jnp.exp(-jnp.sum(usage * jnp.log(usage + 1e-10),
                                        axis=1, keepdims=True))
        usage_ref[...] = usage


@functools.partial(jax.jit, static_argnames=("interpret",))
def kernel(ze, codebook, interpret=False):
    b, c, h, w = ze.shape
    flat = jnp.transpose(ze, (0, 2, 3, 1)).reshape(-1, c)

    out_shapes = (
        jax.ShapeDtypeStruct((N_TOKENS, DIM), jnp.float32),
        jax.ShapeDtypeStruct((NB, 1, BLK), jnp.int32),
        jax.ShapeDtypeStruct((1, 1), jnp.float32),
        jax.ShapeDtypeStruct((1, NUM_CODES), jnp.float32),
        jax.ShapeDtypeStruct((1, 1), jnp.float32),
        jax.ShapeDtypeStruct((1, 1), jnp.int32),
    )
    zq, idx, loss, usage, ppl, dead = pl.pallas_call(
        _vq_kernel,
        grid=(NB,),
        in_specs=[
            pl.BlockSpec((BLK, DIM), lambda i: (i, 0)),
            pl.BlockSpec((NUM_CODES, DIM), lambda i: (0, 0)),
        ],
        out_specs=(
            pl.BlockSpec((BLK, DIM), lambda i: (i, 0)),
            pl.BlockSpec((1, 1, BLK), lambda i: (i, 0, 0)),
            pl.BlockSpec((1, 1), lambda i: (0, 0)),
            pl.BlockSpec((1, NUM_CODES), lambda i: (0, 0)),
            pl.BlockSpec((1, 1), lambda i: (0, 0)),
            pl.BlockSpec((1, 1), lambda i: (0, 0)),
        ),
        out_shape=out_shapes,
        interpret=interpret,
    )(flat, codebook)

    zq_st = jnp.transpose(zq.reshape(b, h, w, c), (0, 3, 1, 2))
    indices = idx.reshape(b, h, w)
    loss = loss.reshape(())
    ppl = ppl.reshape(())
    dead = dead.reshape(())
    usage = usage.reshape(NUM_CODES)
    return (zq_st, indices, loss, loss, ppl, dead, usage)


# speedup vs baseline: 2.0153x; 1.2545x over previous
import functools

import jax
import jax.numpy as jnp
from jax.experimental import pallas as pl

NUM_CODES = 1024
DIM = 64
DEAD_THRESHOLD = 2.0
N_TOKENS = 16 * 32 * 32
BLK = 4096
NB = N_TOKENS // BLK


def _vq_kernel(flat_ref, cb_ref, zq_ref, idx_ref,
               loss_ref, usage_ref, ppl_ref, dead_ref):
    i = pl.program_id(0)
    nb = pl.num_programs(0)

    flat = flat_ref[...]
    cb = cb_ref[...]
    z2 = jnp.sum(flat * flat, axis=1)
    cb2 = jnp.sum(cb * cb, axis=1)

    zc = jax.lax.dot_general(
        flat, cb, (((1,), (1,)), ((), ())),
        preferred_element_type=jnp.float32,
        precision=jax.lax.Precision.DEFAULT)
    d = (z2[:, None] - 2.0 * zc) + cb2[None, :]

    mind = jnp.min(d, axis=1, keepdims=True)
    cidx = jax.lax.broadcasted_iota(jnp.int32, (1, NUM_CODES), 1)
    idx = jnp.min(jnp.where(d == mind, cidx, NUM_CODES), axis=1)
    idx = idx.astype(jnp.int32)

    oh_bool = cidx == idx[:, None]
    oh_bf = oh_bool.astype(jnp.bfloat16)
    oh_f = oh_bool.astype(jnp.float32)

    cb_hi = cb.astype(jnp.bfloat16)
    zq = jax.lax.dot_general(oh_bf, cb_hi, (((1,), (0,)), ((), ())),
                             preferred_element_type=jnp.float32)
    zq_ref[...] = zq
    idx_ref[0, 0, :] = idx

    bl = jnp.sum(mind).reshape(1, 1)
    uc = jnp.sum(oh_f, axis=0, keepdims=True)

    @pl.when(i == 0)
    def _init():
        loss_ref[...] = bl
        usage_ref[...] = uc

    @pl.when(i != 0)
    def _acc():
        loss_ref[...] += bl
        usage_ref[...] += uc

    @pl.when(i == nb - 1)
    def _finalize():
        counts = usage_ref[...]
        loss_ref[...] = loss_ref[...] * (1.0 / (N_TOKENS * DIM))
        dead_ref[...] = jnp.sum((counts < DEAD_THRESHOLD).astype(jnp.int32),
                                axis=1, keepdims=True)
        usage = counts * (1.0 / N_TOKENS)
        ppl_ref[...] = jnp.exp(-jnp.sum(usage * jnp.log(usage + 1e-10),
                                        axis=1, keepdims=True))
        usage_ref[...] = usage


@functools.partial(jax.jit, static_argnames=("interpret",))
def kernel(ze, codebook, interpret=False):
    b, c, h, w = ze.shape
    flat = jnp.transpose(ze, (0, 2, 3, 1)).reshape(-1, c)

    out_shapes = (
        jax.ShapeDtypeStruct((N_TOKENS, DIM), jnp.float32),
        jax.ShapeDtypeStruct((NB, 1, BLK), jnp.int32),
        jax.ShapeDtypeStruct((1, 1), jnp.float32),
        jax.ShapeDtypeStruct((1, NUM_CODES), jnp.float32),
        jax.ShapeDtypeStruct((1, 1), jnp.float32),
        jax.ShapeDtypeStruct((1, 1), jnp.int32),
    )
    zq, idx, loss, usage, ppl, dead = pl.pallas_call(
        _vq_kernel,
        grid=(NB,),
        in_specs=[
            pl.BlockSpec((BLK, DIM), lambda i: (i, 0)),
            pl.BlockSpec((NUM_CODES, DIM), lambda i: (0, 0)),
        ],
        out_specs=(
            pl.BlockSpec((BLK, DIM), lambda i: (i, 0)),
            pl.BlockSpec((1, 1, BLK), lambda i: (i, 0, 0)),
            pl.BlockSpec((1, 1), lambda i: (0, 0)),
            pl.BlockSpec((1, NUM_CODES), lambda i: (0, 0)),
            pl.BlockSpec((1, 1), lambda i: (0, 0)),
            pl.BlockSpec((1, 1), lambda i: (0, 0)),
        ),
        out_shape=out_shapes,
        interpret=interpret,
    )(flat, codebook)

    zq_st = jnp.transpose(zq.reshape(b, h, w, c), (0, 3, 1, 2))
    indices = idx.reshape(b, h, w)
    loss = loss.reshape(())
    ppl = ppl.reshape(())
    dead = dead.reshape(())
    usage = usage.reshape(NUM_CODES)
    return (zq_st, indices, loss, loss, ppl, dead, usage)
